# baseline (speedup 1.0000x reference)
import jax
import jax.numpy as jnp
from jax.experimental import pallas as pl
from jax.experimental.pallas import tpu as pltpu

NEG_SLOPE = 0.2
LOG2E = 1.4426950408889634
BM = 1024
BC = 256
FP = 128


def _gat_body(h_ref, keep_ref, w_ref, a_ref, b_ref, o_ref,
              hp_ref, src_ref, dst_ref):
    g = pl.program_id(0)
    i = pl.program_id(1)
    n = hp_ref.shape[0]
    f_out = hp_ref.shape[1] - FP
    bm = keep_ref.shape[1]

    @pl.when((g == 0) & (i == 0))
    def _():
        hp_ref[:, f_out:] = jnp.ones((n, FP), jnp.bfloat16)

    @pl.when(i == 0)
    def _():
        hp = jnp.dot(h_ref[0], w_ref[0], preferred_element_type=jnp.float32)
        hp_ref[:, :f_out] = hp.astype(jnp.bfloat16)
        t = jnp.tanh(hp)
        sd = jax.lax.dot_general(
            t, a_ref[0], (((1,), (1,)), ((), ())),
            preferred_element_type=jnp.float32) * LOG2E
        src_ref[...] = sd[:, 0:1]
        dst_ref[...] = jax.lax.dot_general(
            a_ref[0, 1:2, :] * LOG2E, t, (((1,), (1,)), ((), ())),
            preferred_element_type=jnp.float32)

    row0 = i * bm
    dst_row = dst_ref[...]
    e_chunks = []
    for r in range(bm // BC):
        c0 = r * BC
        src_blk = src_ref[pl.ds(row0 + c0, BC), :]
        s = src_blk + dst_row
        s = jnp.maximum(s, NEG_SLOPE * s)
        rows = jax.lax.broadcasted_iota(jnp.int32, (BC, n), 0) + (row0 + c0)
        cols = jax.lax.broadcasted_iota(jnp.int32, (BC, n), 1)
        keep = keep_ref[0, c0:c0 + BC, :] | (rows == cols)
        s = jnp.where(keep, s, -1e30)
        m = jnp.max(s, axis=1, keepdims=True)
        e_chunks.append(jnp.exp2(s - m).astype(jnp.bfloat16))
    e16 = jnp.concatenate(e_chunks, axis=0)
    acc = jnp.dot(e16, hp_ref[...],
                  preferred_element_type=jnp.float32)
    l = acc[:, f_out:f_out + 1]
    o_ref[0, 0] = acc[:, :f_out] * (1.0 / l) + b_ref[...]


def kernel(h, adj, w, a_src, a_dst, bias):
    b, n, f_in = h.shape
    hh, _, f_out = w.shape
    a_cat = jnp.concatenate(
        [a_src[:, :, 0][:, None, :], a_dst[:, :, 0][:, None, :]], axis=1)
    bias2 = bias.reshape(1, f_out)

    grid = (b * hh, n // BM)
    out = pl.pallas_call(
        _gat_body,
        out_shape=jax.ShapeDtypeStruct((b, hh, n, f_out), jnp.float32),
        grid=grid,
        in_specs=[
            pl.BlockSpec((1, n, f_in), lambda g, i: (g // hh, 0, 0)),
            pl.BlockSpec((1, BM, n), lambda g, i: (g // hh, i, 0)),
            pl.BlockSpec((1, f_in, f_out), lambda g, i: (g % hh, 0, 0)),
            pl.BlockSpec((1, 2, f_out), lambda g, i: (g % hh, 0, 0)),
            pl.BlockSpec((1, f_out), lambda g, i: (0, 0)),
        ],
        out_specs=pl.BlockSpec(
            (1, 1, BM, f_out), lambda g, i: (g // hh, g % hh, i, 0)),
        scratch_shapes=[
            pltpu.VMEM((n, f_out + FP), jnp.bfloat16),
            pltpu.VMEM((n, 1), jnp.float32),
            pltpu.VMEM((1, n), jnp.float32),
        ],
        compiler_params=pltpu.CompilerParams(
            dimension_semantics=("parallel", "arbitrary"),
            vmem_limit_bytes=58 * 1024 * 1024,
        ),
        name="fused_graph_attention",
    )(h, adj, w, a_cat, bias2)
    return out


# speedup vs baseline: 1.0658x; 1.0658x over previous
import jax
import jax.numpy as jnp
from jax.experimental import pallas as pl
from jax.experimental.pallas import tpu as pltpu

NEG_SLOPE = 0.2
LOG2E = 1.4426950408889634
BM = 1024
BC = 256
FP = 128


def _gat_body(h_ref, keep_ref, w_ref, a_ref, b_ref, o_ref,
              hp_ref, src_ref, dst_ref):
    g = pl.program_id(0)
    i = pl.program_id(1)
    n = hp_ref.shape[0]
    f_out = hp_ref.shape[1] - FP
    bm = keep_ref.shape[1]

    @pl.when((g == 0) & (i == 0))
    def _():
        hp_ref[:, f_out:] = jnp.ones((n, FP), jnp.bfloat16)

    @pl.when(i == 0)
    def _():
        hp = jnp.dot(h_ref[0], w_ref[0], preferred_element_type=jnp.float32)
        hp_ref[:, :f_out] = hp.astype(jnp.bfloat16)
        t = jnp.tanh(hp)
        sd = jax.lax.dot_general(
            t, a_ref[0], (((1,), (1,)), ((), ())),
            preferred_element_type=jnp.float32) * LOG2E
        src_ref[...] = sd[:, 0:1]
        dst_ref[...] = jax.lax.dot_general(
            a_ref[0, 1:2, :] * LOG2E, t, (((1,), (1,)), ((), ())),
            preferred_element_type=jnp.float32)

    row0 = i * bm
    dst16 = dst_ref[...].astype(jnp.bfloat16)

    def _weights(c0):
        src_blk = src_ref[pl.ds(row0 + c0, BC), :]
        s = src_blk.astype(jnp.bfloat16) + dst16
        s = jnp.maximum(s, jnp.bfloat16(NEG_SLOPE) * s)
        rows = jax.lax.broadcasted_iota(jnp.int32, (BC, n), 0) + (row0 + c0)
        cols = jax.lax.broadcasted_iota(jnp.int32, (BC, n), 1)
        keep = keep_ref[0, c0:c0 + BC, :] | (rows == cols)
        s = jnp.where(keep, s, jnp.bfloat16(-1e30))
        m = jnp.max(s, axis=1, keepdims=True)
        return jnp.exp2(s - m)

    for r in range(bm // (2 * BC)):
        c0 = 2 * r * BC
        e2 = jnp.concatenate([_weights(c0), _weights(c0 + BC)], axis=0)
        acc = jnp.dot(e2, hp_ref[...],
                      preferred_element_type=jnp.float32)
        l = acc[:, f_out:f_out + 1]
        o_ref[0, 0, c0:c0 + 2 * BC, :] = (acc[:, :f_out] * (1.0 / l)
                                          + b_ref[...])


def kernel(h, adj, w, a_src, a_dst, bias):
    b, n, f_in = h.shape
    hh, _, f_out = w.shape
    a_cat = jnp.concatenate(
        [a_src[:, :, 0][:, None, :], a_dst[:, :, 0][:, None, :]], axis=1)
    bias2 = bias.reshape(1, f_out)

    grid = (b * hh, n // BM)
    out = pl.pallas_call(
        _gat_body,
        out_shape=jax.ShapeDtypeStruct((b, hh, n, f_out), jnp.float32),
        grid=grid,
        in_specs=[
            pl.BlockSpec((1, n, f_in), lambda g, i: (g // hh, 0, 0)),
            pl.BlockSpec((1, BM, n), lambda g, i: (g // hh, i, 0)),
            pl.BlockSpec((1, f_in, f_out), lambda g, i: (g % hh, 0, 0)),
            pl.BlockSpec((1, 2, f_out), lambda g, i: (g % hh, 0, 0)),
            pl.BlockSpec((1, f_out), lambda g, i: (0, 0)),
        ],
        out_specs=pl.BlockSpec(
            (1, 1, BM, f_out), lambda g, i: (g // hh, g % hh, i, 0)),
        scratch_shapes=[
            pltpu.VMEM((n, f_out + FP), jnp.bfloat16),
            pltpu.VMEM((n, 1), jnp.float32),
            pltpu.VMEM((1, n), jnp.float32),
        ],
        compiler_params=pltpu.CompilerParams(
            dimension_semantics=("parallel", "arbitrary"),
            vmem_limit_bytes=58 * 1024 * 1024,
        ),
        name="fused_graph_attention",
    )(h, adj, w, a_cat, bias2)
    return out
